# baseline (speedup 1.0000x reference)
import functools

import jax
import jax.numpy as jnp
from jax import lax
from jax.experimental import pallas as pl
from jax.experimental.pallas import tpu as pltpu
from jax.experimental.pallas import tpu_sc as plsc

N = 10000
E = 320000
NPAD = 10240
EPAD = 327680
NCH = EPAD // 128
CPT16 = NCH // 16
CPT32 = NCH // 32
RPT = NPAD // 16
W = 128
BN = 1024
GRID = NPAD // BN


def _sc_mesh():
    return plsc.VectorSubcoreMesh(core_axis_name="c", subcore_axis_name="s")


BLK = 16


def _edge_walk(table, acc, src2d, dst2d, base, nblocks, scr):
    (is0, is1, id0, id1, rows0, rows1, semi0, semi1, gsem0, gsem1) = scr
    ibufs = [(is0, id0, semi0), (is1, id1, semi1)]

    def issue_idx(b):
        ib_s, ib_d, sem = ibufs[b % 2]
        sl = pl.ds(base + b * BLK, BLK)
        pltpu.async_copy(src2d.at[sl], ib_s, sem)
        pltpu.async_copy(dst2d.at[sl], ib_d, sem)

    def wait_idx(b):
        ib_s, ib_d, sem = ibufs[b % 2]
        sl = pl.ds(base + b * BLK, BLK)
        pltpu.make_async_copy(src2d.at[sl], ib_s, sem).wait()
        pltpu.make_async_copy(dst2d.at[sl], ib_d, sem).wait()

    issue_idx(0)
    wait_idx(0)
    pltpu.async_copy(table.at[ibufs[0][0].at[0]], rows0, gsem0)
    for b in range(nblocks):
        ib_s, ib_d, _ = ibufs[b % 2]
        if b + 1 < nblocks:
            issue_idx(b + 1)
        first = b == 0

        @pl.loop(0, BLK, step=2)
        def _(j, ib_s=ib_s, ib_d=ib_d, first=first):
            pltpu.async_copy(table.at[ib_s.at[j + 1]], rows1, gsem1)
            pltpu.make_async_copy(table.at[ib_s.at[j]], rows0, gsem0).wait()
            pltpu.sync_copy(rows0, acc.at[ib_d.at[j]], add=True)

            @pl.when(j + 2 < BLK)
            def _():
                pltpu.async_copy(table.at[ib_s.at[j + 2]], rows0, gsem0)

            pltpu.make_async_copy(table.at[ib_s.at[j + 1]], rows1, gsem1).wait()
            pltpu.sync_copy(rows1, acc.at[ib_d.at[j + 1]], add=True)

        if b + 1 < nblocks:
            wait_idx(b + 1)
            nxt_s = ibufs[(b + 1) % 2][0]
            pltpu.async_copy(table.at[nxt_s.at[0]], rows0, gsem0)


def _make_deg():
    @functools.partial(
        pl.kernel,
        out_type=tuple(
            jax.ShapeDtypeStruct((NPAD, W), jnp.float32) for _ in range(4)),
        mesh=_sc_mesh(),
        scratch_types=[
            pltpu.VMEM((CPT32, 128), jnp.int32),
            pltpu.VMEM((CPT32, 128), jnp.int32),
            pltpu.VMEM((128, W), jnp.float32),
            pltpu.VMEM_SHARED((NPAD, W), jnp.float32),
        ],
    )
    def deg(src2d, dst2d, ones_h, zrows, out_s0, out_s1, out_d0, out_d1,
            idx_s, idx_d, ones_v, hist):
        c = lax.axis_index("c")
        s = lax.axis_index("s")
        wid = c * 16 + s
        pltpu.sync_copy(src2d.at[pl.ds(wid * CPT32, CPT32)], idx_s)
        pltpu.sync_copy(dst2d.at[pl.ds(wid * CPT32, CPT32)], idx_d)
        pltpu.sync_copy(ones_h, ones_v)
        sl = pl.ds(s * RPT, RPT)
        pltpu.sync_copy(zrows, hist.at[sl])
        plsc.subcore_barrier()

        @pl.loop(0, CPT32)
        def _(j):
            pltpu.sync_copy(ones_v, hist.at[idx_s.at[j]], add=True)

        plsc.subcore_barrier()

        @pl.when(c == 0)
        def _():
            pltpu.sync_copy(hist.at[sl], out_s0.at[sl])

        @pl.when(c == 1)
        def _():
            pltpu.sync_copy(hist.at[sl], out_s1.at[sl])

        pltpu.sync_copy(zrows, hist.at[sl])
        plsc.subcore_barrier()

        @pl.loop(0, CPT32)
        def _(j):
            pltpu.sync_copy(ones_v, hist.at[idx_d.at[j]], add=True)

        plsc.subcore_barrier()

        @pl.when(c == 0)
        def _():
            pltpu.sync_copy(hist.at[sl], out_d0.at[sl])

        @pl.when(c == 1)
        def _():
            pltpu.sync_copy(hist.at[sl], out_d1.at[sl])

    return deg


def _agg_scratch(row_shape, dtype):
    return [
        pltpu.VMEM((BLK, 128), jnp.int32),
        pltpu.VMEM((BLK, 128), jnp.int32),
        pltpu.VMEM((BLK, 128), jnp.int32),
        pltpu.VMEM((BLK, 128), jnp.int32),
        pltpu.VMEM((128,) + row_shape, dtype),
        pltpu.VMEM((128,) + row_shape, dtype),
        pltpu.VMEM_SHARED((NPAD,) + row_shape, dtype),
        pltpu.SemaphoreType.DMA,
        pltpu.SemaphoreType.DMA,
        pltpu.SemaphoreType.DMA,
        pltpu.SemaphoreType.DMA,
    ]


def _make_agg_edge_split(row_shape=(W,), dtype=jnp.float32):
    @functools.partial(
        pl.kernel,
        out_type=(
            jax.ShapeDtypeStruct((NPAD,) + row_shape, dtype),
            jax.ShapeDtypeStruct((NPAD,) + row_shape, dtype),
        ),
        mesh=_sc_mesh(),
        scratch_types=_agg_scratch(row_shape, dtype),
    )
    def agg(xa, xb, src2d, dst2d, zrows, out0, out1,
            is0, is1, id0, id1, rows0, rows1, acc,
            semi0, semi1, gsem0, gsem1):
        c = lax.axis_index("c")
        s = lax.axis_index("s")
        wid = c * 16 + s
        sl = pl.ds(s * RPT, RPT)
        pltpu.sync_copy(zrows, acc.at[sl])
        plsc.subcore_barrier()
        scr = (is0, is1, id0, id1, rows0, rows1, semi0, semi1, gsem0, gsem1)

        @pl.when(c == 0)
        def _():
            _edge_walk(xa, acc, src2d, dst2d, wid * CPT32, CPT32 // BLK, scr)

        @pl.when(c == 1)
        def _():
            _edge_walk(xb, acc, src2d, dst2d, wid * CPT32, CPT32 // BLK, scr)

        plsc.subcore_barrier()

        @pl.when(c == 0)
        def _():
            pltpu.sync_copy(acc.at[sl], out0.at[sl])

        @pl.when(c == 1)
        def _():
            pltpu.sync_copy(acc.at[sl], out1.at[sl])

    return agg


def _make_agg_feat_split():
    @functools.partial(
        pl.kernel,
        out_type=(
            jax.ShapeDtypeStruct((NPAD, W), jnp.float32),
            jax.ShapeDtypeStruct((NPAD, W), jnp.float32),
        ),
        mesh=_sc_mesh(),
        scratch_types=_agg_scratch((W,), jnp.float32),
    )
    def agg(x0, x1, src2d, dst2d, zrows, out0, out1,
            is0, is1, id0, id1, rows0, rows1, acc,
            semi0, semi1, gsem0, gsem1):
        c = lax.axis_index("c")
        s = lax.axis_index("s")
        sl = pl.ds(s * RPT, RPT)
        pltpu.sync_copy(zrows, acc.at[sl])
        plsc.subcore_barrier()
        scr = (is0, is1, id0, id1, rows0, rows1, semi0, semi1, gsem0, gsem1)

        @pl.when(c == 0)
        def _():
            _edge_walk(x0, acc, src2d, dst2d, s * CPT16, CPT16 // BLK, scr)

        @pl.when(c == 1)
        def _():
            _edge_walk(x1, acc, src2d, dst2d, s * CPT16, CPT16 // BLK, scr)

        plsc.subcore_barrier()

        @pl.when(c == 0)
        def _():
            pltpu.sync_copy(acc.at[sl], out0.at[sl])

        @pl.when(c == 1)
        def _():
            pltpu.sync_copy(acc.at[sl], out1.at[sl])

    return agg


def _row_spec(w):
    return pl.BlockSpec((BN, w), lambda i: (i, 0))


def _full_spec(shape):
    nd = len(shape)
    return pl.BlockSpec(shape, lambda i: (0,) * nd)


def _make_tc0():
    def body(h_ref, s0, s1, d0, d1, ya_ref, yb_ref, dout_ref, din_ref):
        i = pl.program_id(0)
        rows = lax.broadcasted_iota(jnp.int32, (BN, 1), 0) + i * BN
        mask = rows < N
        degs = s0[...][:, :1] + s1[...][:, :1]
        degd = d0[...][:, :1] + d1[...][:, :1]
        dout = jnp.where(mask, lax.rsqrt(jnp.maximum(degs, 1.0)), 0.0)
        din = jnp.where(mask, lax.rsqrt(jnp.maximum(degd, 1.0)), 0.0)
        dout_ref[...] = dout
        din_ref[...] = din
        ya_ref[...] = h_ref[...] * dout
        yb_ref[...] = h_ref[...] * dout

    return pl.pallas_call(
        body,
        grid=(GRID,),
        in_specs=[_row_spec(128)] * 5,
        out_specs=(_row_spec(128), _row_spec(128),
                   _row_spec(1), _row_spec(1)),
        out_shape=(
            jax.ShapeDtypeStruct((NPAD, 128), jnp.float32),
            jax.ShapeDtypeStruct((NPAD, 128), jnp.float32),
            jax.ShapeDtypeStruct((NPAD, 1), jnp.float32),
            jax.ShapeDtypeStruct((NPAD, 1), jnp.float32),
        ),
    )


def _make_boundary(in_mode, win_shape, blen, wout_shape, out_mode,
                   pad_out=0):
    def body(*refs):
        g0, g1, din, dout = refs[0:4]
        k = 4
        win = None
        if win_shape is not None:
            win = refs[k]
            k += 1
        b = refs[k]
        k += 1
        wout = None
        if wout_shape is not None:
            wout = refs[k]
            k += 1
        outs = refs[k:]
        if in_mode == "sum":
            t = (g0[...] + g1[...]) * din[...]
        else:
            t = jnp.concatenate([g0[...], g1[...]], axis=1) * din[...]
        if win is not None:
            t = jnp.dot(t, win[...], preferred_element_type=jnp.float32)
        x = jnp.maximum(t + b[...][None, :], 0.0)
        y = x * dout[...]
        if wout is not None:
            y = jnp.dot(y, wout[...], preferred_element_type=jnp.float32)
        if pad_out:
            y = jnp.concatenate(
                [y, jnp.zeros((BN, pad_out), jnp.float32)], axis=1)
        if out_mode == "single":
            outs[0][...] = y
            outs[1][...] = y
        else:
            outs[0][...] = y[:, :128]
            outs[1][...] = y[:, 128:]

    in_specs = [_row_spec(128), _row_spec(128), _row_spec(1), _row_spec(1)]
    if win_shape is not None:
        in_specs.append(_full_spec(win_shape))
    in_specs.append(_full_spec((blen,)))
    if wout_shape is not None:
        in_specs.append(_full_spec(wout_shape))
    out_specs = (_row_spec(128), _row_spec(128))
    out_shape = tuple(jax.ShapeDtypeStruct((NPAD, 128), jnp.float32)
                      for _ in range(2))
    return pl.pallas_call(
        body,
        grid=(GRID,),
        in_specs=in_specs,
        out_specs=out_specs,
        out_shape=out_shape,
    )


def _make_tc_final():
    def body(g0, g1, din, w6_ref, b_ref, wc_ref, bc_ref, o_ref, acc_ref):
        i = pl.program_id(0)
        rows = lax.broadcasted_iota(jnp.int32, (BN, 1), 0) + i * BN
        mask = rows < N
        t = ((g0[...] + g1[...]) * din[...])[:, :64]
        x = jnp.dot(t, w6_ref[...], preferred_element_type=jnp.float32)
        x = jnp.maximum(x + b_ref[...][None, :], 0.0)
        x = jnp.where(mask, x, 0.0)
        p = jnp.sum(x, axis=0, keepdims=True)

        @pl.when(i == 0)
        def _():
            acc_ref[...] = p

        @pl.when(i > 0)
        def _():
            acc_ref[...] += p

        @pl.when(i == GRID - 1)
        def _():
            sm = acc_ref[...] * (1.0 / N)
            o_ref[...] = (jnp.dot(sm, wc_ref[...],
                                  preferred_element_type=jnp.float32)
                          + bc_ref[...][None, :])

    return pl.pallas_call(
        body,
        grid=(GRID,),
        in_specs=[_row_spec(128), _row_spec(128), _row_spec(1),
                  _full_spec((64, 32)), _full_spec((32,)),
                  _full_spec((32, 10)), _full_spec((10,))],
        out_specs=pl.BlockSpec((1, 10), lambda i: (0, 0)),
        out_shape=jax.ShapeDtypeStruct((1, 10), jnp.float32),
        scratch_shapes=[pltpu.VMEM((1, 32), jnp.float32)],
    )


_deg_k = _make_deg()
_agg_es = _make_agg_edge_split()
_agg_fs = _make_agg_feat_split()
_tc0_k = _make_tc0()
_tcb1 = _make_boundary("sum", (128, 256), 256, None, "split")
_tcb2 = _make_boundary("concat", (256, 512), 512, (512, 256), "split")
_tcb3 = _make_boundary("concat", None, 256, (256, 128), "single")
_tcb4 = _make_boundary("sum", None, 128, None, "single")
_tcb5 = _make_boundary("sum", (128, 64), 64, None, "single", pad_out=64)
_tcf_k = _make_tc_final()


def kernel(h, edge_index, W1, b1, W2, b2, W3, b3, W4, b4, W5, b5, W6, b6, Wc, bc):
    src = edge_index[0].astype(jnp.int32)
    dst = edge_index[1].astype(jnp.int32)
    fill = jnp.full((EPAD - E,), N, dtype=jnp.int32)
    src2d = jnp.concatenate([src, fill]).reshape(NCH, 128)
    dst2d = jnp.concatenate([dst, fill]).reshape(NCH, 128)
    hp = jnp.pad(h, ((0, NPAD - N), (0, 0)))
    zrows = jnp.zeros((RPT, W), jnp.float32)

    ones = jnp.ones((128, W), jnp.float32)
    s0, s1, d0, d1 = _deg_k(src2d, dst2d, ones, zrows)
    y1a, y1b, dout, din = _tc0_k(hp, s0, s1, d0, d1)

    p0, p1 = _agg_es(y1a, y1b, src2d, dst2d, zrows)
    a0, a1 = _tcb1(p0, p1, din, dout, W1, b1)
    p0, p1 = _agg_fs(a0, a1, src2d, dst2d, zrows)
    a0, a1 = _tcb2(p0, p1, din, dout, W2, b2, W3)
    p0, p1 = _agg_fs(a0, a1, src2d, dst2d, zrows)
    y4a, y4b = _tcb3(p0, p1, din, dout, b3, W4)
    p0, p1 = _agg_es(y4a, y4b, src2d, dst2d, zrows)
    y5a, y5b = _tcb4(p0, p1, din, dout, b4)
    p0, p1 = _agg_es(y5a, y5b, src2d, dst2d, zrows)
    y6a, y6b = _tcb5(p0, p1, din, dout, W5, b5)
    p0, p1 = _agg_es(y6a, y6b, src2d, dst2d, zrows)
    return _tcf_k(p0, p1, din, W6, b6, Wc, bc)


# speedup vs baseline: 1.2125x; 1.2125x over previous
import functools

import jax
import jax.numpy as jnp
from jax import lax
from jax.experimental import pallas as pl
from jax.experimental.pallas import tpu as pltpu
from jax.experimental.pallas import tpu_sc as plsc

N = 10000
E = 320000
NPAD = 10240
EPAD = 327680
NCH = EPAD // 128
CPT16 = NCH // 16
CPT32 = NCH // 32
RPT = NPAD // 16
W = 128
BN = 1024
GRID = NPAD // BN


def _sc_mesh():
    return plsc.VectorSubcoreMesh(core_axis_name="c", subcore_axis_name="s")


BLK = 16


def _edge_walk(table, acc, src2d, dst2d, base, nblocks, scr):
    (is0, is1, id0, id1, rows0, rows1, semi0, semi1, gsem0, gsem1) = scr
    ibufs = [(is0, id0, semi0), (is1, id1, semi1)]

    def issue_idx(b):
        ib_s, ib_d, sem = ibufs[b % 2]
        sl = pl.ds(base + b * BLK, BLK)
        pltpu.async_copy(src2d.at[sl], ib_s, sem)
        pltpu.async_copy(dst2d.at[sl], ib_d, sem)

    def wait_idx(b):
        ib_s, ib_d, sem = ibufs[b % 2]
        sl = pl.ds(base + b * BLK, BLK)
        pltpu.make_async_copy(src2d.at[sl], ib_s, sem).wait()
        pltpu.make_async_copy(dst2d.at[sl], ib_d, sem).wait()

    issue_idx(0)
    wait_idx(0)
    pltpu.async_copy(table.at[ibufs[0][0].at[0]], rows0, gsem0)
    for b in range(nblocks):
        ib_s, ib_d, _ = ibufs[b % 2]
        if b + 1 < nblocks:
            issue_idx(b + 1)
        first = b == 0

        @pl.loop(0, BLK, step=2)
        def _(j, ib_s=ib_s, ib_d=ib_d, first=first):
            pltpu.async_copy(table.at[ib_s.at[j + 1]], rows1, gsem1)
            pltpu.make_async_copy(table.at[ib_s.at[j]], rows0, gsem0).wait()
            pltpu.sync_copy(rows0, acc.at[ib_d.at[j]], add=True)

            @pl.when(j + 2 < BLK)
            def _():
                pltpu.async_copy(table.at[ib_s.at[j + 2]], rows0, gsem0)

            pltpu.make_async_copy(table.at[ib_s.at[j + 1]], rows1, gsem1).wait()
            pltpu.sync_copy(rows1, acc.at[ib_d.at[j + 1]], add=True)

        if b + 1 < nblocks:
            wait_idx(b + 1)
            nxt_s = ibufs[(b + 1) % 2][0]
            pltpu.async_copy(table.at[nxt_s.at[0]], rows0, gsem0)


def _make_deg():
    @functools.partial(
        pl.kernel,
        out_type=tuple(
            jax.ShapeDtypeStruct((NPAD, W), jnp.float32) for _ in range(4)),
        mesh=_sc_mesh(),
        scratch_types=[
            pltpu.VMEM((CPT32, 128), jnp.int32),
            pltpu.VMEM((CPT32, 128), jnp.int32),
            pltpu.VMEM((128, W), jnp.float32),
            pltpu.VMEM_SHARED((NPAD, W), jnp.float32),
        ],
    )
    def deg(src2d, dst2d, ones_h, zrows, out_s0, out_s1, out_d0, out_d1,
            idx_s, idx_d, ones_v, hist):
        c = lax.axis_index("c")
        s = lax.axis_index("s")
        wid = c * 16 + s
        pltpu.sync_copy(src2d.at[pl.ds(wid * CPT32, CPT32)], idx_s)
        pltpu.sync_copy(dst2d.at[pl.ds(wid * CPT32, CPT32)], idx_d)
        pltpu.sync_copy(ones_h, ones_v)
        sl = pl.ds(s * RPT, RPT)
        pltpu.sync_copy(zrows, hist.at[sl])
        plsc.subcore_barrier()

        @pl.loop(0, CPT32)
        def _(j):
            pltpu.sync_copy(ones_v, hist.at[idx_s.at[j]], add=True)

        plsc.subcore_barrier()

        @pl.when(c == 0)
        def _():
            pltpu.sync_copy(hist.at[sl], out_s0.at[sl])

        @pl.when(c == 1)
        def _():
            pltpu.sync_copy(hist.at[sl], out_s1.at[sl])

        pltpu.sync_copy(zrows, hist.at[sl])
        plsc.subcore_barrier()

        @pl.loop(0, CPT32)
        def _(j):
            pltpu.sync_copy(ones_v, hist.at[idx_d.at[j]], add=True)

        plsc.subcore_barrier()

        @pl.when(c == 0)
        def _():
            pltpu.sync_copy(hist.at[sl], out_d0.at[sl])

        @pl.when(c == 1)
        def _():
            pltpu.sync_copy(hist.at[sl], out_d1.at[sl])

    return deg


def _agg_scratch(row_shape, dtype):
    return [
        pltpu.VMEM((BLK, 128), jnp.int32),
        pltpu.VMEM((BLK, 128), jnp.int32),
        pltpu.VMEM((BLK, 128), jnp.int32),
        pltpu.VMEM((BLK, 128), jnp.int32),
        pltpu.VMEM((128,) + row_shape, dtype),
        pltpu.VMEM((128,) + row_shape, dtype),
        pltpu.VMEM_SHARED((NPAD,) + row_shape, dtype),
        pltpu.SemaphoreType.DMA,
        pltpu.SemaphoreType.DMA,
        pltpu.SemaphoreType.DMA,
        pltpu.SemaphoreType.DMA,
    ]


def _make_agg_edge_split(row_shape=(W,), dtype=jnp.float32):
    @functools.partial(
        pl.kernel,
        out_type=(
            jax.ShapeDtypeStruct((NPAD,) + row_shape, dtype),
            jax.ShapeDtypeStruct((NPAD,) + row_shape, dtype),
        ),
        mesh=_sc_mesh(),
        scratch_types=_agg_scratch(row_shape, dtype),
    )
    def agg(x, src2d, dst2d, zrows, out0, out1,
            is0, is1, id0, id1, rows0, rows1, acc,
            semi0, semi1, gsem0, gsem1):
        c = lax.axis_index("c")
        s = lax.axis_index("s")
        wid = c * 16 + s
        sl = pl.ds(s * RPT, RPT)
        pltpu.sync_copy(zrows, acc.at[sl])
        plsc.subcore_barrier()
        scr = (is0, is1, id0, id1, rows0, rows1, semi0, semi1, gsem0, gsem1)
        _edge_walk(x, acc, src2d, dst2d, wid * CPT32, CPT32 // BLK, scr)
        plsc.subcore_barrier()

        @pl.when(c == 0)
        def _():
            pltpu.sync_copy(acc.at[sl], out0.at[sl])

        @pl.when(c == 1)
        def _():
            pltpu.sync_copy(acc.at[sl], out1.at[sl])

    return agg


def _make_agg_feat_split():
    @functools.partial(
        pl.kernel,
        out_type=(
            jax.ShapeDtypeStruct((NPAD, W), jnp.float32),
            jax.ShapeDtypeStruct((NPAD, W), jnp.float32),
        ),
        mesh=_sc_mesh(),
        scratch_types=_agg_scratch((W,), jnp.float32),
    )
    def agg(x0, x1, src2d, dst2d, zrows, out0, out1,
            is0, is1, id0, id1, rows0, rows1, acc,
            semi0, semi1, gsem0, gsem1):
        c = lax.axis_index("c")
        s = lax.axis_index("s")
        sl = pl.ds(s * RPT, RPT)
        pltpu.sync_copy(zrows, acc.at[sl])
        plsc.subcore_barrier()
        scr = (is0, is1, id0, id1, rows0, rows1, semi0, semi1, gsem0, gsem1)

        @pl.when(c == 0)
        def _():
            _edge_walk(x0, acc, src2d, dst2d, s * CPT16, CPT16 // BLK, scr)

        @pl.when(c == 1)
        def _():
            _edge_walk(x1, acc, src2d, dst2d, s * CPT16, CPT16 // BLK, scr)

        plsc.subcore_barrier()

        @pl.when(c == 0)
        def _():
            pltpu.sync_copy(acc.at[sl], out0.at[sl])

        @pl.when(c == 1)
        def _():
            pltpu.sync_copy(acc.at[sl], out1.at[sl])

    return agg


def _row_spec(w):
    return pl.BlockSpec((BN, w), lambda i: (i, 0))


def _full_spec(shape):
    nd = len(shape)
    return pl.BlockSpec(shape, lambda i: (0,) * nd)


def _make_tc0():
    def body(h_ref, s0, s1, d0, d1, y_ref, dout_ref, din_ref):
        i = pl.program_id(0)
        rows = lax.broadcasted_iota(jnp.int32, (BN, 1), 0) + i * BN
        mask = rows < N
        degs = s0[...][:, :1] + s1[...][:, :1]
        degd = d0[...][:, :1] + d1[...][:, :1]
        dout = jnp.where(mask, lax.rsqrt(jnp.maximum(degs, 1.0)), 0.0)
        din = jnp.where(mask, lax.rsqrt(jnp.maximum(degd, 1.0)), 0.0)
        dout_ref[...] = dout
        din_ref[...] = din
        y_ref[...] = h_ref[...] * dout

    return pl.pallas_call(
        body,
        grid=(GRID,),
        in_specs=[_row_spec(128)] * 5,
        out_specs=(_row_spec(128), _row_spec(1), _row_spec(1)),
        out_shape=(
            jax.ShapeDtypeStruct((NPAD, 128), jnp.float32),
            jax.ShapeDtypeStruct((NPAD, 1), jnp.float32),
            jax.ShapeDtypeStruct((NPAD, 1), jnp.float32),
        ),
    )


def _make_boundary(in_mode, win_shape, blen, wout_shape, out_mode,
                   pad_out=0):
    def body(*refs):
        g0, g1, din, dout = refs[0:4]
        k = 4
        win = None
        if win_shape is not None:
            win = refs[k]
            k += 1
        b = refs[k]
        k += 1
        wout = None
        if wout_shape is not None:
            wout = refs[k]
            k += 1
        outs = refs[k:]
        if in_mode == "sum":
            t = (g0[...] + g1[...]) * din[...]
        else:
            t = jnp.concatenate([g0[...], g1[...]], axis=1) * din[...]
        if win is not None:
            t = jnp.dot(t, win[...], preferred_element_type=jnp.float32)
        x = jnp.maximum(t + b[...][None, :], 0.0)
        y = x * dout[...]
        if wout is not None:
            y = jnp.dot(y, wout[...], preferred_element_type=jnp.float32)
        if pad_out:
            y = jnp.concatenate(
                [y, jnp.zeros((BN, pad_out), jnp.float32)], axis=1)
        if out_mode == "single":
            outs[0][...] = y
        else:
            outs[0][...] = y[:, :128]
            outs[1][...] = y[:, 128:]

    in_specs = [_row_spec(128), _row_spec(128), _row_spec(1), _row_spec(1)]
    if win_shape is not None:
        in_specs.append(_full_spec(win_shape))
    in_specs.append(_full_spec((blen,)))
    if wout_shape is not None:
        in_specs.append(_full_spec(wout_shape))
    n_out = 1 if out_mode == "single" else 2
    out_specs = tuple(_row_spec(128) for _ in range(n_out))
    out_shape = tuple(jax.ShapeDtypeStruct((NPAD, 128), jnp.float32)
                      for _ in range(n_out))
    return pl.pallas_call(
        body,
        grid=(GRID,),
        in_specs=in_specs,
        out_specs=out_specs,
        out_shape=out_shape,
    )


def _make_tc_final():
    def body(g0, g1, din, w6_ref, b_ref, wc_ref, bc_ref, o_ref, acc_ref):
        i = pl.program_id(0)
        rows = lax.broadcasted_iota(jnp.int32, (BN, 1), 0) + i * BN
        mask = rows < N
        t = ((g0[...] + g1[...]) * din[...])[:, :64]
        x = jnp.dot(t, w6_ref[...], preferred_element_type=jnp.float32)
        x = jnp.maximum(x + b_ref[...][None, :], 0.0)
        x = jnp.where(mask, x, 0.0)
        p = jnp.sum(x, axis=0, keepdims=True)

        @pl.when(i == 0)
        def _():
            acc_ref[...] = p

        @pl.when(i > 0)
        def _():
            acc_ref[...] += p

        @pl.when(i == GRID - 1)
        def _():
            sm = acc_ref[...] * (1.0 / N)
            o_ref[...] = (jnp.dot(sm, wc_ref[...],
                                  preferred_element_type=jnp.float32)
                          + bc_ref[...][None, :])

    return pl.pallas_call(
        body,
        grid=(GRID,),
        in_specs=[_row_spec(128), _row_spec(128), _row_spec(1),
                  _full_spec((64, 32)), _full_spec((32,)),
                  _full_spec((32, 10)), _full_spec((10,))],
        out_specs=pl.BlockSpec((1, 10), lambda i: (0, 0)),
        out_shape=jax.ShapeDtypeStruct((1, 10), jnp.float32),
        scratch_shapes=[pltpu.VMEM((1, 32), jnp.float32)],
    )


_deg_k = _make_deg()
_agg_es = _make_agg_edge_split()
_agg_fs = _make_agg_feat_split()
_tc0_k = _make_tc0()
_tcb1 = _make_boundary("sum", (128, 256), 256, None, "split")
_tcb2 = _make_boundary("concat", (256, 512), 512, (512, 256), "split")
_tcb3 = _make_boundary("concat", None, 256, (256, 128), "single")
_tcb4 = _make_boundary("sum", None, 128, None, "single")
_tcb5 = _make_boundary("sum", (128, 64), 64, None, "single", pad_out=64)
_tcf_k = _make_tc_final()


def kernel(h, edge_index, W1, b1, W2, b2, W3, b3, W4, b4, W5, b5, W6, b6, Wc, bc):
    src = edge_index[0].astype(jnp.int32)
    dst = edge_index[1].astype(jnp.int32)
    fill = jnp.full((EPAD - E,), N, dtype=jnp.int32)
    src2d = jnp.concatenate([src, fill]).reshape(NCH, 128)
    dst2d = jnp.concatenate([dst, fill]).reshape(NCH, 128)
    hp = jnp.pad(h, ((0, NPAD - N), (0, 0)))
    zrows = jnp.zeros((RPT, W), jnp.float32)

    ones = jnp.ones((128, W), jnp.float32)
    s0, s1, d0, d1 = _deg_k(src2d, dst2d, ones, zrows)
    y1, dout, din = _tc0_k(hp, s0, s1, d0, d1)

    p0, p1 = _agg_es(y1, src2d, dst2d, zrows)
    a0, a1 = _tcb1(p0, p1, din, dout, W1, b1)
    p0, p1 = _agg_fs(a0, a1, src2d, dst2d, zrows)
    a0, a1 = _tcb2(p0, p1, din, dout, W2, b2, W3)
    p0, p1 = _agg_fs(a0, a1, src2d, dst2d, zrows)
    y4, = _tcb3(p0, p1, din, dout, b3, W4)
    p0, p1 = _agg_es(y4, src2d, dst2d, zrows)
    y5, = _tcb4(p0, p1, din, dout, b4)
    p0, p1 = _agg_es(y5, src2d, dst2d, zrows)
    y6, = _tcb5(p0, p1, din, dout, W5, b5)
    p0, p1 = _agg_es(y6, src2d, dst2d, zrows)
    return _tcf_k(p0, p1, din, W6, b6, Wc, bc)
